# baseline (speedup 1.0000x reference)
import functools

import jax
import jax.numpy as jnp
from jax import lax
from jax.experimental import pallas as pl
from jax.experimental.pallas import tpu as pltpu
from jax.experimental.pallas import tpu_sc as plsc

_L = 16


def _make_sc_gather_max(N, K, D, NC, NS, C=32, interpret=False):
    NW = NC * NS
    CK = C * K
    T = -(-N // (NW * C))
    Npad = NW * T * C
    VL = 2 * _L

    def body(y_hbm, idx_hbm, out_hbm, idx_v, rows_v, out_v, sem):
        wid = lax.axis_index("s") * NC + lax.axis_index("c")
        pltpu.sync_copy(idx_hbm.at[wid], idx_v)
        base = wid * (T * C)

        def chunk(t, carry):
            pltpu.async_copy(y_hbm.at[idx_v.at[t]], rows_v, sem).wait()

            def node(c, carry2):
                r0 = c * K
                for d in range(D // VL):
                    sl = pl.ds(d * VL, VL)
                    vals = [rows_v[r0 + k, sl] for k in range(K)]
                    while len(vals) > 1:
                        vals = [jnp.maximum(vals[j], vals[j + 1])
                                for j in range(0, len(vals) - 1, 2)] + (
                                    [vals[-1]] if len(vals) % 2 else [])
                    out_v[c, sl] = vals[0]
                return carry2

            lax.fori_loop(0, C, node, 0)
            pltpu.sync_copy(out_v, out_hbm.at[pl.ds(base + t * C, C)])
            return carry

        lax.fori_loop(0, T, chunk, 0)

    mesh = plsc.VectorSubcoreMesh(core_axis_name="c", subcore_axis_name="s",
                                  num_cores=NC, num_subcores=NS)
    gm = pl.kernel(
        body,
        out_type=jax.ShapeDtypeStruct((Npad, D), jnp.bfloat16),
        mesh=mesh,
        scratch_types=[
            pltpu.VMEM((T, CK), jnp.int32),
            pltpu.VMEM((CK, D), jnp.bfloat16),
            pltpu.VMEM((C, D), jnp.bfloat16),
            pltpu.SemaphoreType.DMA,
        ],
        compiler_params=pltpu.CompilerParams(use_tc_tiling_on_sc=False),
        interpret=interpret,
    )
    return gm, NW, T, C, Npad


def _mm_body(x_ref, w_ref, o_ref):
    o_ref[...] = jnp.dot(x_ref[...], w_ref[...],
                         preferred_element_type=jnp.float32).astype(jnp.bfloat16)


def _combine_mid_body(x_ref, agg_ref, ws_ref, wa_ref, b_ref, bp_ref, g_ref,
                      beta_ref, wpn_ref, out_ref, y_ref):
    agg = jnp.maximum(agg_ref[...].astype(jnp.float32) + bp_ref[0], 0.0)
    h = jnp.dot(x_ref[...].astype(jnp.float32), ws_ref[...],
                preferred_element_type=jnp.float32)
    h = h + jnp.dot(agg, wa_ref[...], preferred_element_type=jnp.float32)
    h = h + b_ref[0]
    mu = jnp.mean(h, axis=-1, keepdims=True)
    var = jnp.mean((h - mu) ** 2, axis=-1, keepdims=True)
    ln = (h - mu) * lax.rsqrt(var + 1e-5) * g_ref[0] + beta_ref[0]
    h = jnp.maximum(ln, 0.0) + h
    out_ref[...] = h.astype(jnp.bfloat16)
    y_ref[...] = jnp.dot(h, wpn_ref[...],
                         preferred_element_type=jnp.float32).astype(jnp.bfloat16)


def _combine_last_body(x_ref, agg_ref, ws_ref, wa_ref, b_ref, bp_ref, out_ref):
    agg = jnp.maximum(agg_ref[...].astype(jnp.float32) + bp_ref[0], 0.0)
    h = jnp.dot(x_ref[...].astype(jnp.float32), ws_ref[...],
                preferred_element_type=jnp.float32)
    h = h + jnp.dot(agg, wa_ref[...], preferred_element_type=jnp.float32)
    out_ref[...] = h + b_ref[0]


def _row_spec(BN, D):
    return pl.BlockSpec((BN, D), lambda i: (i, 0))


def _full_spec(shape):
    return pl.BlockSpec(shape, lambda i: tuple(0 for _ in shape))


def _tc_matmul(x, w, BN):
    N, D = x.shape
    Dout = w.shape[1]
    return pl.pallas_call(
        _mm_body,
        grid=(N // BN,),
        in_specs=[_row_spec(BN, D), _full_spec(w.shape)],
        out_specs=_row_spec(BN, Dout),
        out_shape=jax.ShapeDtypeStruct((N, Dout), jnp.bfloat16),
    )(x, w)


def _tc_combine_mid(x, agg_pad, ws, wa, b, bp, g, beta, wpn, BN):
    N, D = x.shape
    grid = (N // BN,)
    out_shape = [
        jax.ShapeDtypeStruct((N, D), jnp.bfloat16),
        jax.ShapeDtypeStruct((N, D), jnp.bfloat16),
    ]
    return pl.pallas_call(
        _combine_mid_body,
        grid=grid,
        in_specs=[
            _row_spec(BN, D),
            _row_spec(BN, D),
            _full_spec(ws.shape),
            _full_spec(wa.shape),
            _full_spec(b.shape),
            _full_spec(bp.shape),
            _full_spec(g.shape),
            _full_spec(beta.shape),
            _full_spec(wpn.shape),
        ],
        out_specs=[_row_spec(BN, D), _row_spec(BN, D)],
        out_shape=out_shape,
    )(x, agg_pad, ws, wa, b, bp, g, beta, wpn)


def _tc_combine_last(x, agg_pad, ws, wa, b, bp, BN):
    N, D = x.shape
    Dout = ws.shape[1]
    return pl.pallas_call(
        _combine_last_body,
        grid=(N // BN,),
        in_specs=[
            _row_spec(BN, D),
            _row_spec(BN, D),
            _full_spec(ws.shape),
            _full_spec(wa.shape),
            _full_spec(b.shape),
            _full_spec(bp.shape),
        ],
        out_specs=_row_spec(BN, Dout),
        out_shape=jax.ShapeDtypeStruct((N, Dout), jnp.float32),
    )(x, agg_pad, ws, wa, b, bp)


def kernel(features, neighbors, Wp0, bp0, Wp1, bp1, Wp2, bp2,
           W0, b0, W1, b1, W2, b2, g0, beta0, g1, beta1):
    N, D = features.shape
    K = neighbors.shape[1]
    try:
        info = plsc.get_sparse_core_info()
        nc, ns = info.num_cores, info.num_subcores
    except Exception:
        nc, ns = 2, 16
    H = 1
    Nh = N // H
    gm, NW, T, Cn, Npad = _make_sc_gather_max(Nh, K, D, nc, ns)
    BN = next((c for c in (2000, 1000, 1024, 512, 256, 8) if Nh % c == 0), Nh)

    idx = neighbors.astype(jnp.int32).reshape(H, Nh * K)
    idxs = [
        jnp.pad(idx[j], (0, Npad * K - Nh * K)).reshape(NW, T, -1)
        for j in range(H)
    ]

    r2 = lambda v: v.reshape(1, -1)
    pools = [(Wp0, r2(bp0)), (Wp1, r2(bp1)), (Wp2, r2(bp2))]
    lins = [(W0, r2(b0)), (W1, r2(b1)), (W2, r2(b2))]
    norms = [(r2(g0), r2(beta0)), (r2(g1), r2(beta1))]

    xs = [features[j * Nh:(j + 1) * Nh] for j in range(H)]
    y = _tc_matmul(features, Wp0, BN)
    for i in range(3):
        aggs = [gm(y, idxs[j]) for j in range(H)]
        W, b = lins[i]
        ws, wa = W[:D], W[D:]
        bp = pools[i][1]
        if i < 2:
            g, beta = norms[i]
            wpn = pools[i + 1][0]
            res = [_tc_combine_mid(xs[j], aggs[j], ws, wa, b, bp, g, beta,
                                   wpn, BN) for j in range(H)]
            xs = [r[0] for r in res]
            ys = [r[1] for r in res]
            y = ys[0] if H == 1 else jnp.concatenate(ys, axis=0)
        else:
            xs = [_tc_combine_last(xs[j], aggs[j], ws, wa, b, bp, BN)
                  for j in range(H)]
    return xs[0] if H == 1 else jnp.concatenate(xs, axis=0)


# speedup vs baseline: 1.2172x; 1.2172x over previous
import functools

import jax
import jax.numpy as jnp
from jax import lax
from jax.experimental import pallas as pl
from jax.experimental.pallas import tpu as pltpu
from jax.experimental.pallas import tpu_sc as plsc

_L = 16


def _make_sc_gather_max(N, K, D, NC, NS, C=32, interpret=False):
    NW = NC * NS
    CK = C * K
    T = -(-N // (NW * C))
    Npad = NW * T * C
    VL = 2 * _L

    def body(y_hbm, idx_hbm, out_hbm, idx_v, rows0, rows1, out_v, sem0, sem1):
        wid = lax.axis_index("s") * NC + lax.axis_index("c")
        pltpu.sync_copy(idx_hbm.at[wid], idx_v)
        base = wid * (T * C)
        rows = (rows0, rows1)
        sems = (sem0, sem1)

        def phase(t, b):
            @pl.when(t + 1 < T)
            def _():
                pltpu.async_copy(y_hbm.at[idx_v.at[t + 1]], rows[1 - b],
                                 sems[1 - b])

            pltpu.make_async_copy(y_hbm.at[idx_v.at[t]], rows[b],
                                  sems[b]).wait()
            rv = rows[b]

            def node(c, carry2):
                r0 = c * K
                for d in range(D // VL):
                    sl = pl.ds(d * VL, VL)
                    acc = rv[r0, sl]
                    for k in range(1, K):
                        acc = jnp.maximum(acc, rv[r0 + k, sl])
                    out_v[c, sl] = acc
                return carry2

            lax.fori_loop(0, C, node, 0)
            pltpu.sync_copy(out_v, out_hbm.at[pl.ds(base + t * C, C)])

        pltpu.async_copy(y_hbm.at[idx_v.at[0]], rows0, sem0)

        def two(i, carry):
            phase(2 * i, 0)
            phase(2 * i + 1, 1)
            return carry

        lax.fori_loop(0, T // 2, two, 0)
        for tt in range(2 * (T // 2), T):
            phase(tt, tt % 2)

    mesh = plsc.VectorSubcoreMesh(core_axis_name="c", subcore_axis_name="s",
                                  num_cores=NC, num_subcores=NS)
    gm = pl.kernel(
        body,
        out_type=jax.ShapeDtypeStruct((Npad, D), jnp.bfloat16),
        mesh=mesh,
        scratch_types=[
            pltpu.VMEM((T, CK), jnp.int32),
            pltpu.VMEM((CK, D), jnp.bfloat16),
            pltpu.VMEM((CK, D), jnp.bfloat16),
            pltpu.VMEM((C, D), jnp.bfloat16),
            pltpu.SemaphoreType.DMA,
            pltpu.SemaphoreType.DMA,
        ],
        compiler_params=pltpu.CompilerParams(use_tc_tiling_on_sc=False),
        interpret=interpret,
    )
    return gm, NW, T, C, Npad


def _mm_body(x_ref, w_ref, o_ref):
    o_ref[...] = jnp.dot(x_ref[...], w_ref[...],
                         preferred_element_type=jnp.float32).astype(jnp.bfloat16)


def _combine_mid_body(x_ref, agg_ref, ws_ref, wa_ref, b_ref, bp_ref, g_ref,
                      beta_ref, wpn_ref, out_ref, y_ref):
    agg = jnp.maximum(agg_ref[...].astype(jnp.float32) + bp_ref[0], 0.0)
    h = jnp.dot(x_ref[...].astype(jnp.float32), ws_ref[...],
                preferred_element_type=jnp.float32)
    h = h + jnp.dot(agg, wa_ref[...], preferred_element_type=jnp.float32)
    h = h + b_ref[0]
    mu = jnp.mean(h, axis=-1, keepdims=True)
    var = jnp.mean((h - mu) ** 2, axis=-1, keepdims=True)
    ln = (h - mu) * lax.rsqrt(var + 1e-5) * g_ref[0] + beta_ref[0]
    h = jnp.maximum(ln, 0.0) + h
    out_ref[...] = h.astype(jnp.bfloat16)
    y_ref[...] = jnp.dot(h, wpn_ref[...],
                         preferred_element_type=jnp.float32).astype(jnp.bfloat16)


def _combine_last_body(x_ref, agg_ref, ws_ref, wa_ref, b_ref, bp_ref, out_ref):
    agg = jnp.maximum(agg_ref[...].astype(jnp.float32) + bp_ref[0], 0.0)
    h = jnp.dot(x_ref[...].astype(jnp.float32), ws_ref[...],
                preferred_element_type=jnp.float32)
    h = h + jnp.dot(agg, wa_ref[...], preferred_element_type=jnp.float32)
    out_ref[...] = h + b_ref[0]


def _row_spec(BN, D):
    return pl.BlockSpec((BN, D), lambda i: (i, 0))


def _full_spec(shape):
    return pl.BlockSpec(shape, lambda i: tuple(0 for _ in shape))


def _tc_matmul(x, w, BN):
    N, D = x.shape
    Dout = w.shape[1]
    return pl.pallas_call(
        _mm_body,
        grid=(N // BN,),
        in_specs=[_row_spec(BN, D), _full_spec(w.shape)],
        out_specs=_row_spec(BN, Dout),
        out_shape=jax.ShapeDtypeStruct((N, Dout), jnp.bfloat16),
    )(x, w)


def _tc_combine_mid(x, agg_pad, ws, wa, b, bp, g, beta, wpn, BN):
    N, D = x.shape
    grid = (N // BN,)
    out_shape = [
        jax.ShapeDtypeStruct((N, D), jnp.bfloat16),
        jax.ShapeDtypeStruct((N, D), jnp.bfloat16),
    ]
    return pl.pallas_call(
        _combine_mid_body,
        grid=grid,
        in_specs=[
            _row_spec(BN, D),
            _row_spec(BN, D),
            _full_spec(ws.shape),
            _full_spec(wa.shape),
            _full_spec(b.shape),
            _full_spec(bp.shape),
            _full_spec(g.shape),
            _full_spec(beta.shape),
            _full_spec(wpn.shape),
        ],
        out_specs=[_row_spec(BN, D), _row_spec(BN, D)],
        out_shape=out_shape,
    )(x, agg_pad, ws, wa, b, bp, g, beta, wpn)


def _tc_combine_last(x, agg_pad, ws, wa, b, bp, BN):
    N, D = x.shape
    Dout = ws.shape[1]
    return pl.pallas_call(
        _combine_last_body,
        grid=(N // BN,),
        in_specs=[
            _row_spec(BN, D),
            _row_spec(BN, D),
            _full_spec(ws.shape),
            _full_spec(wa.shape),
            _full_spec(b.shape),
            _full_spec(bp.shape),
        ],
        out_specs=_row_spec(BN, Dout),
        out_shape=jax.ShapeDtypeStruct((N, Dout), jnp.float32),
    )(x, agg_pad, ws, wa, b, bp)


def kernel(features, neighbors, Wp0, bp0, Wp1, bp1, Wp2, bp2,
           W0, b0, W1, b1, W2, b2, g0, beta0, g1, beta1):
    N, D = features.shape
    K = neighbors.shape[1]
    try:
        info = plsc.get_sparse_core_info()
        nc, ns = info.num_cores, info.num_subcores
    except Exception:
        nc, ns = 2, 16
    H = 1
    Nh = N // H
    gm, NW, T, Cn, Npad = _make_sc_gather_max(Nh, K, D, nc, ns)
    BN = next((c for c in (2000, 1000, 1024, 512, 256, 8) if Nh % c == 0), Nh)

    idx = neighbors.astype(jnp.int32).reshape(H, Nh * K)
    idxs = [
        jnp.pad(idx[j], (0, Npad * K - Nh * K)).reshape(NW, T, -1)
        for j in range(H)
    ]

    r2 = lambda v: v.reshape(1, -1)
    pools = [(Wp0, r2(bp0)), (Wp1, r2(bp1)), (Wp2, r2(bp2))]
    lins = [(W0, r2(b0)), (W1, r2(b1)), (W2, r2(b2))]
    norms = [(r2(g0), r2(beta0)), (r2(g1), r2(beta1))]

    xs = [features[j * Nh:(j + 1) * Nh] for j in range(H)]
    y = _tc_matmul(features, Wp0, BN)
    for i in range(3):
        aggs = [gm(y, idxs[j]) for j in range(H)]
        W, b = lins[i]
        ws, wa = W[:D], W[D:]
        bp = pools[i][1]
        if i < 2:
            g, beta = norms[i]
            wpn = pools[i + 1][0]
            res = [_tc_combine_mid(xs[j], aggs[j], ws, wa, b, bp, g, beta,
                                   wpn, BN) for j in range(H)]
            xs = [r[0] for r in res]
            ys = [r[1] for r in res]
            y = ys[0] if H == 1 else jnp.concatenate(ys, axis=0)
        else:
            xs = [_tc_combine_last(xs[j], aggs[j], ws, wa, b, bp, BN)
                  for j in range(H)]
    return xs[0] if H == 1 else jnp.concatenate(xs, axis=0)
